# baseline (speedup 1.0000x reference)
import functools

import jax
import jax.numpy as jnp
from jax import lax
from jax.experimental import pallas as pl
from jax.experimental.pallas import tpu as pltpu
from jax.experimental.pallas import tpu_sc as plsc

HIDDEN_DIM = 2048
NUM_EXPERTS = 16
N_TOKENS = 16384
BLK = 2048

_NC, _NS = 2, 16
_NW = _NC * _NS
_NCHUNKS = 4
_CHUNK = N_TOKENS // _NCHUNKS
_TPW = _CHUNK // _NW


def _logits_block(x_ref, w_ref, b_ref, logits_ref):
    logits_ref[...] = lax.dot_general(
        x_ref[...], w_ref[...],
        (((1,), (1,)), ((), ())),
        preferred_element_type=jnp.float32,
    ) + b_ref[...]


def _tc_logits(x, W, b):
    n_tokens = x.shape[0]
    return pl.pallas_call(
        _logits_block,
        grid=(n_tokens // BLK,),
        in_specs=[
            pl.BlockSpec((BLK, HIDDEN_DIM), lambda i: (i, 0)),
            pl.BlockSpec((NUM_EXPERTS, HIDDEN_DIM), lambda i: (0, 0)),
            pl.BlockSpec((1, NUM_EXPERTS), lambda i: (0, 0)),
        ],
        out_specs=pl.BlockSpec((BLK, NUM_EXPERTS), lambda i: (i, 0)),
        out_shape=jax.ShapeDtypeStruct((n_tokens, NUM_EXPERTS), jnp.float32),
    )(x, W, b.reshape(1, NUM_EXPERTS))


_mesh = plsc.VectorSubcoreMesh(core_axis_name="c", subcore_axis_name="s")


@functools.partial(
    pl.kernel,
    mesh=_mesh,
    compiler_params=pltpu.CompilerParams(needs_layout_passes=False),
    out_type=[
        jax.ShapeDtypeStruct((_CHUNK * NUM_EXPERTS,), jnp.float32),
        jax.ShapeDtypeStruct((_CHUNK * 2,), jnp.int32),
    ],
    scratch_types=[
        pltpu.VMEM((_TPW * NUM_EXPERTS,), jnp.float32),
        pltpu.VMEM((_TPW * NUM_EXPERTS,), jnp.float32),
        pltpu.VMEM((_TPW * 2 + NUM_EXPERTS,), jnp.int32),
    ],
)
def _sc_tail(logits_hbm, gtop_hbm, idx_hbm, l_v, g_v, i_v):
    wid = lax.axis_index("s") * _NC + lax.axis_index("c")
    base = wid * _TPW
    pltpu.sync_copy(logits_hbm.at[pl.ds(base * NUM_EXPERTS, _TPW * NUM_EXPERTS)], l_v)

    iota = lax.iota(jnp.int32, NUM_EXPERTS)

    def lane(v, j):
        idx = jnp.full((NUM_EXPERTS, 1), j, jnp.int32)
        dnums = lax.GatherDimensionNumbers(
            offset_dims=(), collapsed_slice_dims=(0,), start_index_map=(0,))
        return lax.gather(v, idx, dnums, (1,),
                          mode=lax.GatherScatterMode.PROMISE_IN_BOUNDS)

    @plsc.parallel_loop(0, _TPW, 1, unroll=8)
    def body(t):
        l = l_v[pl.ds(t * NUM_EXPERTS, NUM_EXPERTS)]
        sk, sv = plsc.sort_key_val(l, iota, descending=True)
        mb = lane(sk, 0)
        v2b = lane(sk, 1)
        i1b = lane(sv, 0)
        i2b = lane(sv, 1)

        e = jnp.exp(l - mb)
        sb = lane(plsc.cumsum(e), NUM_EXPERTS - 1)
        g1 = 1.0 / sb
        g2 = jnp.exp(v2b - mb) / sb
        denom = g1 + g2 + 1e-9
        g_v[pl.ds(t * NUM_EXPERTS, NUM_EXPERTS)] = jnp.where(iota == i1b, g1 / denom,
                              jnp.where(iota == i2b, g2 / denom, 0.0))
        pair = jnp.where(iota == 0, i1b, i2b)
        plsc.store_compressed(i_v.at[pl.ds(2 * t, NUM_EXPERTS)], pair, mask=iota < 2)

    pltpu.sync_copy(g_v, gtop_hbm.at[pl.ds(base * NUM_EXPERTS, _TPW * NUM_EXPERTS)])
    pltpu.sync_copy(i_v.at[pl.ds(0, _TPW * 2)], idx_hbm.at[pl.ds(base * 2, _TPW * 2)])


def kernel(x, W, b):
    logits_chunks = [
        _tc_logits(lax.slice_in_dim(x, c * _CHUNK, (c + 1) * _CHUNK), W, b)
        for c in range(_NCHUNKS)
    ]
    tails = [_sc_tail(lc.reshape(_CHUNK * NUM_EXPERTS)) for lc in logits_chunks]
    g_top = jnp.concatenate(
        [g.reshape(_CHUNK, NUM_EXPERTS) for g, _ in tails], axis=0)
    idx = jnp.concatenate([i.reshape(_CHUNK, 2) for _, i in tails], axis=0)
    logits = jnp.concatenate(logits_chunks, axis=0)
    return (g_top, idx, logits)


# speedup vs baseline: 2.7787x; 2.7787x over previous
import jax
import jax.numpy as jnp
from jax import lax
from jax.experimental import pallas as pl

HIDDEN_DIM = 2048
NUM_EXPERTS = 16
N_TOKENS = 16384
BLK = 2048


def _router_block(x_ref, w_ref, b_ref, gtop_ref, idx_ref, logits_ref):
    logits = lax.dot_general(
        x_ref[...], w_ref[...],
        (((1,), (1,)), ((), ())),
        preferred_element_type=jnp.float32,
    ) + b_ref[...]
    logits_ref[...] = logits

    m = jnp.max(logits, axis=1, keepdims=True)
    iota = lax.broadcasted_iota(jnp.int32, logits.shape, 1)
    i1 = jnp.min(jnp.where(logits == m, iota, NUM_EXPERTS), axis=1, keepdims=True)
    lm = jnp.where(iota == i1, -jnp.inf, logits)
    v2 = jnp.max(lm, axis=1, keepdims=True)
    i2 = jnp.min(jnp.where(lm == v2, iota, NUM_EXPERTS), axis=1, keepdims=True)

    e = jnp.exp(logits - m)
    s = jnp.sum(e, axis=1, keepdims=True)
    g1 = 1.0 / s
    g2 = jnp.exp(v2 - m) / s
    denom = g1 + g2 + 1e-9
    gtop_ref[...] = jnp.where(iota == i1, g1 / denom,
                              jnp.where(iota == i2, g2 / denom, 0.0))
    idx_ref[...] = jnp.concatenate([i1, i2], axis=1)


def kernel(x, W, b):
    n_tokens = x.shape[0]
    grid = (n_tokens // BLK,)
    g_top, idx, logits = pl.pallas_call(
        _router_block,
        grid=grid,
        in_specs=[
            pl.BlockSpec((BLK, HIDDEN_DIM), lambda i: (i, 0)),
            pl.BlockSpec((NUM_EXPERTS, HIDDEN_DIM), lambda i: (0, 0)),
            pl.BlockSpec((1, NUM_EXPERTS), lambda i: (0, 0)),
        ],
        out_specs=[
            pl.BlockSpec((BLK, NUM_EXPERTS), lambda i: (i, 0)),
            pl.BlockSpec((BLK, 2), lambda i: (i, 0)),
            pl.BlockSpec((BLK, NUM_EXPERTS), lambda i: (i, 0)),
        ],
        out_shape=[
            jax.ShapeDtypeStruct((n_tokens, NUM_EXPERTS), jnp.float32),
            jax.ShapeDtypeStruct((n_tokens, 2), jnp.int32),
            jax.ShapeDtypeStruct((n_tokens, NUM_EXPERTS), jnp.float32),
        ],
    )(x, W, b.reshape(1, NUM_EXPERTS))
    return (g_top, idx, logits)
